# Initial kernel scaffold; baseline (speedup 1.0000x reference)
import jax
import jax.numpy as jnp
from jax.experimental import pallas as pl


def kernel(tokens, table):
    raise NotImplementedError("write your pallas kernel here")


# speedup vs baseline: 4.2875x; 4.2875x over previous
import functools

import jax
import jax.numpy as jnp
from jax import lax
from jax.experimental import pallas as pl
from jax.experimental.pallas import tpu as pltpu
from jax.experimental.pallas import tpu_sc as plsc

VOCAB = 28996
DIM = 32
LANES = 128


def _make_kernel(num_rows: int, rows_per_w: int):
    mesh = plsc.VectorSubcoreMesh(core_axis_name="c", subcore_axis_name="s")
    info = plsc.get_sparse_core_info()
    nc = info.num_cores

    @functools.partial(
        pl.kernel,
        mesh=mesh,
        compiler_params=pltpu.CompilerParams(use_tc_tiling_on_sc=False),
        out_type=jax.ShapeDtypeStruct((num_rows, LANES, DIM), jnp.float32),
        scratch_types=[
            pltpu.VMEM((rows_per_w, LANES), jnp.int32),
            pltpu.VMEM((LANES, DIM), jnp.float32),
            pltpu.SemaphoreType.DMA,
        ],
    )
    def k(tokens_hbm, table_hbm, out_hbm, idx_v, rows_v, sem):
        wid = lax.axis_index("s") * nc + lax.axis_index("c")
        base = wid * rows_per_w
        pltpu.sync_copy(tokens_hbm.at[pl.ds(base, rows_per_w)], idx_v)

        def body(j, _):
            pltpu.async_copy(table_hbm.at[idx_v.at[j]], rows_v, sem).wait()
            pltpu.sync_copy(rows_v, out_hbm.at[base + j])
            return 0

        lax.fori_loop(0, rows_per_w, body, 0)

    return k


def kernel(tokens, table):
    b, t = tokens.shape
    n = b * t
    num_rows = n // LANES
    nw = 32
    rows_per_w = num_rows // nw
    idx = tokens.reshape(num_rows, LANES).astype(jnp.int32)
    out = _make_kernel(num_rows, rows_per_w)(idx, table)
    return out.reshape(b, t, DIM)


# speedup vs baseline: 5.3512x; 1.2481x over previous
import functools

import jax
import jax.numpy as jnp
from jax import lax
from jax.experimental import pallas as pl
from jax.experimental.pallas import tpu as pltpu
from jax.experimental.pallas import tpu_sc as plsc

DIM = 32
LANES = 128
K = 10
SLAB = K * LANES


def _make_kernel(num_rows: int, rows_per_w: int):
    mesh = plsc.VectorSubcoreMesh(core_axis_name="c", subcore_axis_name="s")
    info = plsc.get_sparse_core_info()
    nc = info.num_cores
    num_chunks = rows_per_w // K
    assert num_chunks % 2 == 0 and num_chunks * K == rows_per_w
    n_super = num_chunks // 2

    @functools.partial(
        pl.kernel,
        mesh=mesh,
        compiler_params=pltpu.CompilerParams(use_tc_tiling_on_sc=False),
        out_type=jax.ShapeDtypeStruct((num_rows * LANES, DIM), jnp.float32),
        scratch_types=[
            pltpu.VMEM((rows_per_w, LANES), jnp.int32),
            pltpu.VMEM((SLAB, DIM), jnp.float32),
            pltpu.VMEM((SLAB, DIM), jnp.float32),
            pltpu.SemaphoreType.DMA,
            pltpu.SemaphoreType.DMA,
            pltpu.SemaphoreType.DMA,
            pltpu.SemaphoreType.DMA,
        ],
    )
    def k(tokens_hbm, table_hbm, out_hbm, idx_v, buf_a, buf_b, ga, gb, wa, wb):
        wid = lax.axis_index("s") * nc + lax.axis_index("c")
        base_row = wid * rows_per_w
        tok0 = base_row * LANES
        pltpu.sync_copy(tokens_hbm.at[pl.ds(base_row, rows_per_w)], idx_v)

        def fire(c, buf, gsem):
            hs = []
            for b in range(K):
                hs.append(
                    pltpu.async_copy(
                        table_hbm.at[idx_v.at[c * K + b]],
                        buf.at[pl.ds(b * LANES, LANES)],
                        gsem,
                    )
                )
            return hs

        def body(s, _):
            ca = 2 * s
            cb = 2 * s + 1

            @pl.when(s > 0)
            def _():
                pltpu.make_async_copy(buf_a, out_hbm.at[pl.ds(0, SLAB)], wa).wait()

            ha = fire(ca, buf_a, ga)

            @pl.when(s > 0)
            def _():
                pltpu.make_async_copy(buf_b, out_hbm.at[pl.ds(0, SLAB)], wb).wait()

            hb = fire(cb, buf_b, gb)
            for h in ha:
                h.wait()
            pltpu.async_copy(buf_a, out_hbm.at[pl.ds(tok0 + ca * SLAB, SLAB)], wa)
            for h in hb:
                h.wait()
            pltpu.async_copy(buf_b, out_hbm.at[pl.ds(tok0 + cb * SLAB, SLAB)], wb)
            return 0

        lax.fori_loop(0, n_super, body, 0)
        pltpu.make_async_copy(buf_a, out_hbm.at[pl.ds(0, SLAB)], wa).wait()
        pltpu.make_async_copy(buf_b, out_hbm.at[pl.ds(0, SLAB)], wb).wait()

    return k


def kernel(tokens, table):
    b, t = tokens.shape
    n = b * t
    num_rows = n // LANES
    nw = 32
    rows_per_w = num_rows // nw
    idx = tokens.reshape(num_rows, LANES).astype(jnp.int32)
    out = _make_kernel(num_rows, rows_per_w)(idx, table)
    return out.reshape(b, t, DIM)


# speedup vs baseline: 5.4262x; 1.0140x over previous
import functools

import jax
import jax.numpy as jnp
from jax import lax
from jax.experimental import pallas as pl
from jax.experimental.pallas import tpu as pltpu
from jax.experimental.pallas import tpu_sc as plsc

DIM = 32
LANES = 128
K = 4
SLAB = K * LANES
NSUB = 16


def _make_kernel(num_rows: int, rows_per_w: int, vpad: int):
    mesh = plsc.VectorSubcoreMesh(core_axis_name="c", subcore_axis_name="s")
    info = plsc.get_sparse_core_info()
    nc = info.num_cores
    num_chunks = rows_per_w // K
    assert num_chunks % 2 == 0 and num_chunks * K == rows_per_w
    n_super = num_chunks // 2
    stage_rows = vpad // NSUB

    @functools.partial(
        pl.kernel,
        mesh=mesh,
        compiler_params=pltpu.CompilerParams(use_tc_tiling_on_sc=False),
        out_type=jax.ShapeDtypeStruct((num_rows * LANES, DIM), jnp.float32),
        scratch_types=[
            pltpu.VMEM_SHARED((vpad, DIM), jnp.float32),
            pltpu.VMEM((rows_per_w, LANES), jnp.int32),
            pltpu.VMEM((SLAB, DIM), jnp.float32),
            pltpu.VMEM((SLAB, DIM), jnp.float32),
            pltpu.SemaphoreType.DMA,
            pltpu.SemaphoreType.DMA,
            pltpu.SemaphoreType.DMA,
            pltpu.SemaphoreType.DMA,
        ],
    )
    def k(tokens_hbm, table_hbm, out_hbm, tab_s, idx_v, buf_a, buf_b, ga, gb, wa, wb):
        cid = lax.axis_index("c")
        sid = lax.axis_index("s")
        wid = sid * nc + cid
        base_row = wid * rows_per_w
        tok0 = base_row * LANES

        hstage = pltpu.async_copy(
            table_hbm.at[pl.ds(sid * stage_rows, stage_rows)],
            tab_s.at[pl.ds(sid * stage_rows, stage_rows)],
            ga,
        )
        pltpu.sync_copy(tokens_hbm.at[pl.ds(base_row, rows_per_w)], idx_v)
        hstage.wait()
        plsc.subcore_barrier()

        def fire(c, buf, gsem):
            hs = []
            for b in range(K):
                hs.append(
                    pltpu.async_copy(
                        tab_s.at[idx_v.at[c * K + b]],
                        buf.at[pl.ds(b * LANES, LANES)],
                        gsem,
                    )
                )
            return hs

        def body(s, _):
            ca = 2 * s
            cb = 2 * s + 1

            @pl.when(s > 0)
            def _():
                pltpu.make_async_copy(buf_a, out_hbm.at[pl.ds(0, SLAB)], wa).wait()

            ha = fire(ca, buf_a, ga)

            @pl.when(s > 0)
            def _():
                pltpu.make_async_copy(buf_b, out_hbm.at[pl.ds(0, SLAB)], wb).wait()

            hb = fire(cb, buf_b, gb)
            for h in ha:
                h.wait()
            pltpu.async_copy(buf_a, out_hbm.at[pl.ds(tok0 + ca * SLAB, SLAB)], wa)
            for h in hb:
                h.wait()
            pltpu.async_copy(buf_b, out_hbm.at[pl.ds(tok0 + cb * SLAB, SLAB)], wb)
            return 0

        lax.fori_loop(0, n_super, body, 0)
        pltpu.make_async_copy(buf_a, out_hbm.at[pl.ds(0, SLAB)], wa).wait()
        pltpu.make_async_copy(buf_b, out_hbm.at[pl.ds(0, SLAB)], wb).wait()

    return k


def kernel(tokens, table):
    b, t = tokens.shape
    n = b * t
    num_rows = n // LANES
    nw = 32
    rows_per_w = num_rows // nw
    idx = tokens.reshape(num_rows, LANES).astype(jnp.int32)
    v = table.shape[0]
    vpad = ((v + NSUB * 8 - 1) // (NSUB * 8)) * (NSUB * 8)
    tab = jnp.pad(table, ((0, vpad - v), (0, 0)))
    out = _make_kernel(num_rows, rows_per_w, vpad)(idx, tab)
    return out.reshape(b, t, DIM)


# speedup vs baseline: 5.4327x; 1.0012x over previous
import functools

import jax
import jax.numpy as jnp
from jax import lax
from jax.experimental import pallas as pl
from jax.experimental.pallas import tpu as pltpu
from jax.experimental.pallas import tpu_sc as plsc

DIM = 32
LANES = 128
K = 5
SLAB = K * LANES
NSUB = 16


def _make_kernel(num_rows: int, rows_per_w: int, vpad: int):
    mesh = plsc.VectorSubcoreMesh(core_axis_name="c", subcore_axis_name="s")
    info = plsc.get_sparse_core_info()
    nc = info.num_cores
    num_chunks = rows_per_w // K
    assert num_chunks % 2 == 0 and num_chunks * K == rows_per_w
    n_super = num_chunks // 2
    stage_rows = vpad // NSUB

    @functools.partial(
        pl.kernel,
        mesh=mesh,
        compiler_params=pltpu.CompilerParams(use_tc_tiling_on_sc=False),
        out_type=jax.ShapeDtypeStruct((num_rows * LANES, DIM), jnp.float32),
        scratch_types=[
            pltpu.VMEM_SHARED((vpad, DIM), jnp.float32),
            pltpu.VMEM((rows_per_w, LANES), jnp.int32),
            pltpu.VMEM((SLAB, DIM), jnp.float32),
            pltpu.VMEM((SLAB, DIM), jnp.float32),
            pltpu.SemaphoreType.DMA,
            pltpu.SemaphoreType.DMA,
            pltpu.SemaphoreType.DMA,
            pltpu.SemaphoreType.DMA,
        ],
    )
    def k(tokens_hbm, table_hbm, out_hbm, tab_s, idx_v, buf_a, buf_b, ga, gb, wa, wb):
        cid = lax.axis_index("c")
        sid = lax.axis_index("s")
        wid = sid * nc + cid
        base_row = wid * rows_per_w
        tok0 = base_row * LANES

        hstage = pltpu.async_copy(
            table_hbm.at[pl.ds(sid * stage_rows, stage_rows)],
            tab_s.at[pl.ds(sid * stage_rows, stage_rows)],
            ga,
        )
        pltpu.sync_copy(tokens_hbm.at[pl.ds(base_row, rows_per_w)], idx_v)
        hstage.wait()
        plsc.subcore_barrier()

        def fire(c, buf, gsem):
            hs = []
            for b in range(K):
                hs.append(
                    pltpu.async_copy(
                        tab_s.at[idx_v.at[c * K + b]],
                        buf.at[pl.ds(b * LANES, LANES)],
                        gsem,
                    )
                )
            return hs

        def body(s, _):
            ca = 2 * s
            cb = 2 * s + 1

            @pl.when(s > 0)
            def _():
                pltpu.make_async_copy(buf_a, out_hbm.at[pl.ds(0, SLAB)], wa).wait()

            ha = fire(ca, buf_a, ga)

            @pl.when(s > 0)
            def _():
                pltpu.make_async_copy(buf_b, out_hbm.at[pl.ds(0, SLAB)], wb).wait()

            hb = fire(cb, buf_b, gb)
            for h in ha:
                h.wait()
            pltpu.async_copy(buf_a, out_hbm.at[pl.ds(tok0 + ca * SLAB, SLAB)], wa)
            for h in hb:
                h.wait()
            pltpu.async_copy(buf_b, out_hbm.at[pl.ds(tok0 + cb * SLAB, SLAB)], wb)
            return 0

        lax.fori_loop(0, n_super, body, 0)
        pltpu.make_async_copy(buf_a, out_hbm.at[pl.ds(0, SLAB)], wa).wait()
        pltpu.make_async_copy(buf_b, out_hbm.at[pl.ds(0, SLAB)], wb).wait()

    return k


def kernel(tokens, table):
    b, t = tokens.shape
    n = b * t
    num_rows = n // LANES
    nw = 32
    rows_per_w = num_rows // nw
    idx = tokens.reshape(num_rows, LANES).astype(jnp.int32)
    v = table.shape[0]
    vpad = ((v + NSUB * 8 - 1) // (NSUB * 8)) * (NSUB * 8)
    tab = jnp.pad(table, ((0, vpad - v), (0, 0)))
    out = _make_kernel(num_rows, rows_per_w, vpad)(idx, tab)
    return out.reshape(b, t, DIM)
